# baseline (speedup 1.0000x reference)
import functools

import jax
import jax.numpy as jnp
from jax import lax
from jax.experimental import pallas as pl
from jax.experimental.pallas import tpu as pltpu
from jax.experimental.pallas import tpu_sc as plsc

_NC = 2
_NS = 16
_NW = _NC * _NS
_LANES = 16
_E = 8 * 1024
_U = 8


def kernel(x, pe):
    B, S, D = x.shape
    xf = x.reshape(B, S * D)
    pef = pe[:S].reshape(S * D)
    per_w = (S * D) // _NW
    n_steps = per_w // _E
    chunks = _E // _LANES

    mesh = plsc.VectorSubcoreMesh(core_axis_name="c", subcore_axis_name="s")

    @functools.partial(
        pl.kernel,
        mesh=mesh,
        out_type=jax.ShapeDtypeStruct((B, S * D), jnp.float32),
        scratch_types=(
            [pltpu.VMEM((_E,), jnp.float32)]
            + [pltpu.VMEM((_E,), jnp.float32) for _ in range(2 * B)]
            + [pltpu.SemaphoreType.DMA]
        ),
    )
    def sc_add(x_hbm, pe_hbm, out_hbm, pe_v, x0, x1, x2, x3, o0, o1, o2, o3, sem):
        xb = [x0, x1, x2, x3]
        ob = [o0, o1, o2, o3]
        wid = lax.axis_index("s") * _NC + lax.axis_index("c")
        base = wid * per_w

        def step(i, _):
            off = base + i * _E
            copies = [pltpu.async_copy(pe_hbm.at[pl.ds(off, _E)], pe_v, sem)]
            for b in range(B):
                copies.append(
                    pltpu.async_copy(x_hbm.at[b, pl.ds(off, _E)], xb[b], sem)
                )
            for c in copies:
                c.wait()

            @plsc.parallel_loop(0, chunks, unroll=_U)
            def add_chunk(j):
                c0 = j * _LANES
                pchunk = pe_v[pl.ds(c0, _LANES)]
                for b in range(B):
                    ob[b][pl.ds(c0, _LANES)] = xb[b][pl.ds(c0, _LANES)] + pchunk

            for b in range(B):
                pltpu.sync_copy(ob[b], out_hbm.at[b, pl.ds(off, _E)])
            return 0

        lax.fori_loop(0, n_steps, step, 0)

    return sc_add(xf, pef).reshape(B, S, D)
